# baseline (speedup 1.0000x reference)
import functools

import jax
import jax.numpy as jnp
from jax import lax
from jax.experimental import pallas as pl
from jax.experimental.pallas import tpu as pltpu
from jax.experimental.pallas import tpu_sc as plsc

B_ = 64
NIMG = 192
C_ = 96
HW = 256
NCOMP = 512
D = C_ * HW

NB = 16

_GW = 24
_NCHUNK = 48
_CR = 4


def _sc_gather(table, idx_pad):
    mesh = plsc.VectorSubcoreMesh(core_axis_name="c", subcore_axis_name="s")

    @functools.partial(
        pl.kernel,
        mesh=mesh,
        out_type=jax.ShapeDtypeStruct((_NCHUNK, _CR, D), jnp.float32),
        scratch_types=[
            pltpu.VMEM((_CR,), jnp.int32),
            pltpu.VMEM((_CR, D), jnp.float32),
            pltpu.SemaphoreType.DMA,
        ],
    )
    def k(table_hbm, idx_hbm, out_hbm, idx_v, rows_v, sem):
        wid = lax.axis_index("s") * 2 + lax.axis_index("c")

        @pl.when(wid < _GW)
        def _():
            for h in range(_NCHUNK // _GW):
                j = wid * 2 + h
                pltpu.sync_copy(idx_hbm.at[pl.ds(8 * j, _CR)], idx_v)
                pltpu.async_copy(table_hbm.at[idx_v], rows_v, sem).wait()
                pltpu.sync_copy(rows_v, out_hbm.at[j])

    return k(table, idx_pad)


def _shift_rows(a, s):
    if s == 0:
        return a
    n, _, c = a.shape
    if s > 0:
        pad = jnp.zeros((n, s, c), a.dtype)
        return jnp.concatenate([a[:, s:, :], pad], axis=1)
    pad = jnp.zeros((n, -s, c), a.dtype)
    return jnp.concatenate([pad, a[:, :HW + s, :]], axis=1)


def _conv_body(g_ref, x_ref, w_ref, b_ref, o_ref):
    nb = g_ref.shape[0]
    a = g_ref[...]
    p = lax.broadcasted_iota(jnp.int32, (HW, 1), 0)
    hh = p // 16
    ww = p % 16
    for l in range(3):
        ab = jnp.maximum(a, 0.0).astype(jnp.bfloat16)
        acc = jnp.zeros((nb, HW, C_), jnp.float32)
        for ky in range(3):
            for kx in range(3):
                s = (ky - 1) * 16 + (kx - 1)
                m = ((hh + (ky - 1) >= 0) & (hh + (ky - 1) < 16)
                     & (ww + (kx - 1) >= 0) & (ww + (kx - 1) < 16))
                piece = jnp.where(m[None], _shift_rows(ab, s),
                                  jnp.bfloat16(0.0))
                acc = acc + jnp.dot(
                    piece.reshape(nb * HW, C_), w_ref[l, ky, kx],
                    preferred_element_type=jnp.float32).reshape(nb, HW, C_)
        a = acc + b_ref[l][None, None, :]
    o_ref[...] = x_ref[...] + a


def _conv_call(gt, xt, wt, bs):
    grid = (NIMG // NB,)
    return pl.pallas_call(
        _conv_body,
        grid=grid,
        in_specs=[
            pl.BlockSpec((NB, HW, C_), lambda i: (i, 0, 0)),
            pl.BlockSpec((NB, HW, C_), lambda i: (i, 0, 0)),
            pl.BlockSpec((3, 3, 3, C_, C_), lambda i: (0, 0, 0, 0, 0)),
            pl.BlockSpec((3, C_), lambda i: (0, 0)),
        ],
        out_specs=pl.BlockSpec((NB, HW, C_), lambda i: (i, 0, 0)),
        out_shape=jax.ShapeDtypeStruct((NIMG, HW, C_), jnp.float32),
    )(gt, xt, wt, bs)


def kernel(x, comp_addrs, bias, W1, b1, W2, b2, W3, b3):
    addrs = comp_addrs.reshape(NIMG).astype(jnp.int32)
    addrs_p = jnp.pad(addrs.reshape(_NCHUNK, _CR),
                      ((0, 0), (0, 8 - _CR))).reshape(_NCHUNK * 8)
    g = _sc_gather(bias.reshape(NCOMP, D), addrs_p)
    gt = g.reshape(NIMG, C_, HW).transpose(0, 2, 1)
    xt = x.reshape(NIMG, C_, HW).transpose(0, 2, 1)
    wt = jnp.stack([W1, W2, W3]).transpose(0, 3, 4, 2, 1).astype(jnp.bfloat16)
    bs = jnp.stack([b1, b2, b3])
    yt = _conv_call(gt, xt, wt, bs)
    return yt.transpose(0, 2, 1).reshape(B_, 3, C_, 16, 16)
